# baseline (speedup 1.0000x reference)
import functools

import jax
import jax.numpy as jnp
from jax import lax
from jax.experimental import pallas as pl
from jax.experimental.pallas import tpu as pltpu
from jax.experimental.pallas import tpu_sc as plsc

_ROWS = 1000000
_BATCH = 16384
_DIM = 64
_LANES = 16
_NC = 2
_NS = 16
_NW = _NC * _NS
_NT = (_ROWS + 127) // 128
_SPAN = 245
_T_STEP = _NT - _SPAN

_mesh = plsc.VectorSubcoreMesh(core_axis_name="c", subcore_axis_name="s")


def _embed_gather_body(idx_hbm, table_hbm, out_hbm, idx_v, hit_r, hit_b,
                       act_r, act_b, cols_v, stage_v, sem, sem_sc):
    wid = lax.axis_index("s") * _NC + lax.axis_index("c")
    t0 = (wid * _T_STEP) // (_NW - 1)
    t1 = jnp.where(wid == _NW - 1, _NT, ((wid + 1) * _T_STEP) // (_NW - 1))
    pltpu.sync_copy(idx_hbm, idx_v.at[pl.ds(0, _BATCH)])

    lane = lax.iota(jnp.int32, _LANES)
    f1_vecs = [(16 * k + lane) >> 3 for k in range(4)]
    f2_vecs = [(16 * k + lane) & 7 for k in range(4)]

    def scan_chunk(c, cnt):
        rm1 = idx_v[pl.ds(c * _LANES, _LANES)] - 1
        rt = rm1 >> 7
        m = (rt >= t0) & (rt < t1)
        plsc.store_compressed(hit_r.at[pl.ds(cnt, _LANES)], rm1, mask=m)
        plsc.store_compressed(hit_b.at[pl.ds(cnt, _LANES)], c * _LANES + lane, mask=m)
        return cnt + plsc.all_reduce_population_count(m)[0]

    cnt = jax.lax.fori_loop(0, _BATCH // _LANES, scan_chunk, jnp.int32(0))
    nch = (cnt + _LANES - 1) >> 4

    def fire(c, buf):
        pltpu.async_copy(
            table_hbm.at[:, :, pl.ds(pl.multiple_of((t0 + c) * 128, 128), 128)],
            cols_v.at[buf],
            sem,
        )

    def process(c, buf, sc_issued):
        t = t0 + c
        pltpu.make_async_copy(
            table_hbm.at[:, :, pl.ds(0, 128)], cols_v.at[buf], sem
        ).wait()

        def col_chunk(ch, acnt):
            hr = hit_r[pl.ds(ch * _LANES, _LANES)]
            hb = hit_b[pl.ds(ch * _LANES, _LANES)]
            m = ((ch * _LANES + lane) < cnt) & ((hr >> 7) == t)
            plsc.store_compressed(act_r.at[pl.ds(acnt, _LANES)], hr, mask=m)
            plsc.store_compressed(act_b.at[pl.ds(acnt, _LANES)], hb, mask=m)
            return acnt + plsc.all_reduce_population_count(m)[0]

        acnt = jax.lax.fori_loop(0, nch, col_chunk, jnp.int32(0))

        def group(g, issued):
            ar = act_r[pl.ds(g * _LANES, _LANES)]
            ab = act_b[pl.ds(g * _LANES, _LANES)]
            gm = (g * _LANES + lane) < acnt
            b_out = jnp.where(gm, ab, _BATCH)
            ring = issued & 1

            @pl.when(issued >= 2)
            def _drain_one():
                pltpu.make_async_copy(
                    out_hbm.at[pl.ds(0, _LANES)], stage_v.at[ring], sem_sc
                ).wait()

            for h in range(_LANES):
                wv = jnp.full((_LANES,), ar[h] & 127, jnp.int32)
                for k in range(4):
                    x = plsc.load_gather(cols_v.at[buf], [f1_vecs[k], f2_vecs[k], wv])
                    stage_v[ring, h, pl.ds(16 * k, _LANES)] = x
            pltpu.async_copy(stage_v.at[ring], out_hbm.at[b_out], sem_sc)
            return issued + 1

        return jax.lax.fori_loop(0, (acnt + _LANES - 1) >> 4, group, sc_issued)

    fire(0, 0)

    def pipelined(c, sc_issued):
        fire(c, c & 1)
        return process(c - 1, (c - 1) & 1, sc_issued)

    sc_issued = jax.lax.fori_loop(1, _SPAN, pipelined, jnp.int32(0))
    sc_issued = process(_SPAN - 1, (_SPAN - 1) & 1, sc_issued)

    @pl.when(sc_issued >= 1)
    def _drain_tail1():
        pltpu.make_async_copy(
            out_hbm.at[pl.ds(0, _LANES)], stage_v.at[0], sem_sc
        ).wait()

    @pl.when(sc_issued >= 2)
    def _drain_tail2():
        pltpu.make_async_copy(
            out_hbm.at[pl.ds(0, _LANES)], stage_v.at[1], sem_sc
        ).wait()


def _make_embed_gather(interpret=False):
    return functools.partial(
        pl.kernel,
        mesh=_mesh,
        out_type=jax.ShapeDtypeStruct((_BATCH + 1, 2 * _DIM), jnp.float32),
        scratch_types=[
            pltpu.VMEM((_BATCH + _LANES,), jnp.int32),
            pltpu.VMEM((_BATCH + _LANES,), jnp.int32),
            pltpu.VMEM((_BATCH + _LANES,), jnp.int32),
            pltpu.VMEM((_BATCH + _LANES,), jnp.int32),
            pltpu.VMEM((_BATCH + _LANES,), jnp.int32),
            pltpu.VMEM((2, 8, 8, 128), jnp.float32),
            pltpu.VMEM((2, _LANES, 2 * _DIM), jnp.float32),
            pltpu.SemaphoreType.DMA,
            pltpu.SemaphoreType.DMA,
        ],
        compiler_params=pltpu.CompilerParams(needs_layout_passes=False),
        interpret=interpret,
    )(_embed_gather_body)


_embed_gather = _make_embed_gather()


def kernel(action, embedding):
    table3 = embedding.T.reshape(8, 8, _ROWS)
    out_full = _embed_gather(action.astype(jnp.int32), table3)
    return out_full[:_BATCH, :_DIM]


# speedup vs baseline: 20.0064x; 20.0064x over previous
import functools

import jax
import jax.numpy as jnp
from jax import lax
from jax.experimental import pallas as pl
from jax.experimental.pallas import tpu as pltpu
from jax.experimental.pallas import tpu_sc as plsc

_ROWS = 1000000
_BATCH = 16384
_DIM = 64
_LANES = 16
_NC = 2
_NS = 16
_NW = _NC * _NS
_B_PER_W = _BATCH // _NW
_GSZ = 4
_NG = _B_PER_W // _GSZ

_mesh = plsc.VectorSubcoreMesh(core_axis_name="c", subcore_axis_name="s")


def _embed_gather_body(idx_hbm, table_hbm, out_hbm, idx_v, cols_v, out_v, sem):
    wid = lax.axis_index("s") * _NC + lax.axis_index("c")
    base = wid * _B_PER_W
    pltpu.sync_copy(idx_hbm.at[pl.ds(base, _B_PER_W)], idx_v.at[pl.ds(0, _B_PER_W)])

    lane = lax.iota(jnp.int32, _LANES)
    f1_vecs = [(16 * k + lane) >> 3 for k in range(4)]
    f2_vecs = [(16 * k + lane) & 7 for k in range(4)]

    def fire(g, buf):
        rt = (idx_v[pl.ds(g * _GSZ, _LANES)] - 1) >> 7
        for j in range(_GSZ):
            pltpu.async_copy(
                table_hbm.at[:, :, pl.ds(pl.multiple_of(rt[j] * 128, 128), 128)],
                cols_v.at[buf * _GSZ + j],
                sem,
            )

    def extract(g, buf):
        w = (idx_v[pl.ds(g * _GSZ, _LANES)] - 1) & 127
        for j in range(_GSZ):
            pltpu.make_async_copy(
                table_hbm.at[:, :, pl.ds(0, 128)],
                cols_v.at[buf * _GSZ + j],
                sem,
            ).wait()
        for j in range(_GSZ):
            slot = jnp.full((_LANES,), buf * _GSZ + j, jnp.int32)
            wj = jnp.full((_LANES,), w[j], jnp.int32)
            pos = jnp.full((_LANES,), g * _GSZ + j, jnp.int32)
            for k in range(4):
                x = plsc.load_gather(cols_v, [slot, f1_vecs[k], f2_vecs[k], wj])
                plsc.store_scatter(out_v, [f1_vecs[k], f2_vecs[k], pos], x)

    fire(0, 0)

    def pipelined(g, _):
        fire(g, g & 1)
        extract(g - 1, (g - 1) & 1)
        return _

    jax.lax.fori_loop(1, _NG, pipelined, None)
    extract(_NG - 1, (_NG - 1) & 1)

    for f1 in range(8):
        pltpu.sync_copy(out_v.at[f1], out_hbm.at[f1, :, pl.ds(base, _B_PER_W)])


def _make_embed_gather(interpret=False):
    return functools.partial(
        pl.kernel,
        mesh=_mesh,
        out_type=jax.ShapeDtypeStruct((8, 8, _BATCH), jnp.float32),
        scratch_types=[
            pltpu.VMEM((_B_PER_W + _LANES - _GSZ,), jnp.int32),
            pltpu.VMEM((2 * _GSZ, 8, 8, 128), jnp.float32),
            pltpu.VMEM((8, 8, _B_PER_W), jnp.float32),
            pltpu.SemaphoreType.DMA,
        ],
        compiler_params=pltpu.CompilerParams(needs_layout_passes=False),
        interpret=interpret,
    )(_embed_gather_body)


_embed_gather = _make_embed_gather()


def kernel(action, embedding):
    table3 = embedding.T.reshape(8, 8, _ROWS)
    out3 = _embed_gather(action.astype(jnp.int32), table3)
    return out3.reshape(_DIM, _BATCH).T
